# Initial kernel scaffold; baseline (speedup 1.0000x reference)
import functools

import jax
import jax.numpy as jnp
from jax import lax
from jax.experimental import pallas as pl
from jax.experimental.pallas import tpu as pltpu
from jax.experimental.pallas import tpu_sc as plsc

NNODE = 10000
NP = 10240
NEDGE = 160000
EP = 163840
CHUNK = 128
ROWS_E = EP // CHUNK
NC, NS = 2, 16
RP = NP // NS

_MESH = plsc.VectorSubcoreMesh(core_axis_name="c", subcore_axis_name="s")


def _deg_body(src_h, dst_h, ones_h, zeros_h, outp_h, inp_h,
              sidx, didx, ones_v, shout, shin, sem_a, sem_b):
    c = lax.axis_index("c")
    s = lax.axis_index("s")
    nrows = ROWS_E // (NC * NS)
    base = (c * NS + s) * nrows
    pltpu.sync_copy(src_h.at[pl.ds(base, nrows)], sidx)
    pltpu.sync_copy(dst_h.at[pl.ds(base, nrows)], didx)
    pltpu.sync_copy(ones_h, ones_v)
    pltpu.sync_copy(zeros_h, shout.at[pl.ds(s * RP, RP)])
    pltpu.sync_copy(zeros_h, shin.at[pl.ds(s * RP, RP)])
    plsc.subcore_barrier()

    @pl.loop(0, nrows, step=8)
    def _grp(i):
        hs = []
        for k in range(8):
            hs.append(pltpu.async_copy(ones_v, shout.at[sidx.at[i + k]],
                                       sem_a, add=True))
            hs.append(pltpu.async_copy(ones_v, shin.at[didx.at[i + k]],
                                       sem_b, add=True))
        for h in hs:
            h.wait()

    plsc.subcore_barrier()
    pltpu.sync_copy(shout.at[pl.ds(s * RP, RP)], outp_h.at[c, pl.ds(s * RP, RP)])
    pltpu.sync_copy(shin.at[pl.ds(s * RP, RP)], inp_h.at[c, pl.ds(s * RP, RP)])


_deg_call = functools.partial(
    pl.kernel,
    out_type=[jax.ShapeDtypeStruct((NC, NP, 16), jnp.float32),
              jax.ShapeDtypeStruct((NC, NP, 16), jnp.float32)],
    mesh=_MESH,
    scratch_types=[
        pltpu.VMEM((ROWS_E // (NC * NS), CHUNK), jnp.int32),
        pltpu.VMEM((ROWS_E // (NC * NS), CHUNK), jnp.int32),
        pltpu.VMEM((CHUNK, 16), jnp.float32),
        pltpu.VMEM_SHARED((NP, 16), jnp.float32),
        pltpu.VMEM_SHARED((NP, 16), jnp.float32),
        pltpu.SemaphoreType.DMA,
        pltpu.SemaphoreType.DMA,
    ],
)(_deg_body)


def _make_agg(nfeat, per_sc_edges_split):
    if per_sc_edges_split:
        nrows = ROWS_E // (NC * NS)
    else:
        nrows = ROWS_E // NS
    nbuf = 4

    def body(src_h, dst_h, h_h, zeros_h, agg_h,
             sidx, didx, b0, b1, b2, b3, shared,
             sg0, sg1, sg2, sg3, ss0, ss1, ss2, ss3):
        c = lax.axis_index("c")
        s = lax.axis_index("s")
        bufs = (b0, b1, b2, b3)
        gsem = (sg0, sg1, sg2, sg3)
        ssem = (ss0, ss1, ss2, ss3)
        if per_sc_edges_split:
            base = (c * NS + s) * nrows
            pltpu.sync_copy(src_h.at[pl.ds(base, nrows)], sidx)
        else:
            base = s * nrows
            pltpu.sync_copy(src_h.at[c, pl.ds(base, nrows)], sidx)
        pltpu.sync_copy(dst_h.at[pl.ds(base, nrows)], didx)
        pltpu.sync_copy(zeros_h, shared.at[pl.ds(s * RP, RP)])
        plsc.subcore_barrier()

        @pl.loop(0, nrows, step=nbuf)
        def _grp(i):
            hg = [pltpu.async_copy(h_h.at[sidx.at[i + k]], bufs[k], gsem[k])
                  for k in range(nbuf)]
            hs = []
            for k in range(nbuf):
                hg[k].wait()
                hs.append(pltpu.async_copy(bufs[k], shared.at[didx.at[i + k]],
                                           ssem[k], add=True))
            for h in hs:
                h.wait()

        plsc.subcore_barrier()
        pltpu.sync_copy(shared.at[pl.ds(s * RP, RP)],
                        agg_h.at[c, pl.ds(s * RP, RP)])

    return functools.partial(
        pl.kernel,
        out_type=jax.ShapeDtypeStruct((NC, NP, nfeat), jnp.float32),
        mesh=_MESH,
        scratch_types=[
            pltpu.VMEM((nrows, CHUNK), jnp.int32),
            pltpu.VMEM((nrows, CHUNK), jnp.int32),
            pltpu.VMEM((CHUNK, nfeat), jnp.float32),
            pltpu.VMEM((CHUNK, nfeat), jnp.float32),
            pltpu.VMEM((CHUNK, nfeat), jnp.float32),
            pltpu.VMEM((CHUNK, nfeat), jnp.float32),
            pltpu.VMEM_SHARED((NP, nfeat), jnp.float32),
            pltpu.SemaphoreType.DMA, pltpu.SemaphoreType.DMA,
            pltpu.SemaphoreType.DMA, pltpu.SemaphoreType.DMA,
            pltpu.SemaphoreType.DMA, pltpu.SemaphoreType.DMA,
            pltpu.SemaphoreType.DMA, pltpu.SemaphoreType.DMA,
        ],
    )(body)


_agg1_call = _make_agg(128, per_sc_edges_split=False)
_agg2_call = _make_agg(64, per_sc_edges_split=True)


_PRED_ROWS = (2 * EP) // CHUNK


def _pred_body(u_h, v_h, a_h, b_h, c_h,
               uidx, vidx, b0, b1, b2, b3,
               sa0, sa1, sa2, sa3, sb0, sb1, sb2, sb3,
               so0, so1, so2, so3):
    c = lax.axis_index("c")
    s = lax.axis_index("s")
    nrows = _PRED_ROWS // (NC * NS)
    nbuf = 4
    bufs = (b0, b1, b2, b3)
    asem = (sa0, sa1, sa2, sa3)
    bsem = (sb0, sb1, sb2, sb3)
    osem = (so0, so1, so2, so3)
    base = (c * NS + s) * nrows
    pltpu.sync_copy(u_h.at[pl.ds(base, nrows)], uidx)
    pltpu.sync_copy(v_h.at[pl.ds(base, nrows)], vidx)

    @pl.loop(0, nrows, step=nbuf)
    def _grp(i):
        ha = [pltpu.async_copy(a_h.at[uidx.at[i + k]], bufs[k], asem[k])
              for k in range(nbuf)]
        hb = []
        for k in range(nbuf):
            ha[k].wait()
            hb.append(pltpu.async_copy(b_h.at[vidx.at[i + k]], bufs[k],
                                       bsem[k], add=True))
        ho = []
        for k in range(nbuf):
            hb[k].wait()
            row0 = (base + i + k) * CHUNK
            ho.append(pltpu.async_copy(bufs[k], c_h.at[pl.ds(row0, CHUNK)],
                                       osem[k]))
        for h in ho:
            h.wait()


_pred_call = functools.partial(
    pl.kernel,
    out_type=jax.ShapeDtypeStruct((2 * EP, 64), jnp.float32),
    mesh=_MESH,
    scratch_types=[
        pltpu.VMEM((_PRED_ROWS // (NC * NS), CHUNK), jnp.int32),
        pltpu.VMEM((_PRED_ROWS // (NC * NS), CHUNK), jnp.int32),
        pltpu.VMEM((CHUNK, 64), jnp.float32),
        pltpu.VMEM((CHUNK, 64), jnp.float32),
        pltpu.VMEM((CHUNK, 64), jnp.float32),
        pltpu.VMEM((CHUNK, 64), jnp.float32),
        pltpu.SemaphoreType.DMA, pltpu.SemaphoreType.DMA,
        pltpu.SemaphoreType.DMA, pltpu.SemaphoreType.DMA,
        pltpu.SemaphoreType.DMA, pltpu.SemaphoreType.DMA,
        pltpu.SemaphoreType.DMA, pltpu.SemaphoreType.DMA,
        pltpu.SemaphoreType.DMA, pltpu.SemaphoreType.DMA,
        pltpu.SemaphoreType.DMA, pltpu.SemaphoreType.DMA,
    ],
)(_pred_body)


_RB = 1000


def _rsqrt_deg(ref):
    d = ref[0, :, 0:1] + ref[1, :, 0:1]
    return lax.rsqrt(jnp.maximum(d, 1.0))


def _k2_body(x_ref, w_ref, dego_ref, out_ref):
    ns = _rsqrt_deg(dego_ref)
    xw = jnp.dot(x_ref[...], w_ref[...], preferred_element_type=jnp.float32)
    out_ref[0] = xw * ns


def _k2(x, W1, degout_p):
    return pl.pallas_call(
        _k2_body,
        grid=(2, 10),
        in_specs=[
            pl.BlockSpec((_RB, 512), lambda h, i: (i, 0)),
            pl.BlockSpec((512, 128), lambda h, i: (0, h)),
            pl.BlockSpec((2, _RB, 16), lambda h, i: (0, i, 0)),
        ],
        out_specs=pl.BlockSpec((1, _RB, 128), lambda h, i: (h, i, 0)),
        out_shape=jax.ShapeDtypeStruct((2, NP, 128), jnp.float32),
    )(x, W1, degout_p)


def _k4_body(agg_ref, dego_ref, degi_ref, b1_ref, w2_ref, out_ref):
    ns = _rsqrt_deg(dego_ref)
    nd = _rsqrt_deg(degi_ref)
    a = jnp.maximum(agg_ref[0] * nd + b1_ref[0:1, 0:128], 0.0)
    b = jnp.maximum(agg_ref[1] * nd + b1_ref[0:1, 128:256], 0.0)
    t = (jnp.dot(a, w2_ref[0:128], preferred_element_type=jnp.float32)
         + jnp.dot(b, w2_ref[128:256], preferred_element_type=jnp.float32))
    out_ref[...] = t * ns


def _k4(agg1_p, degout_p, degin_p, b1r, W2):
    return pl.pallas_call(
        _k4_body,
        grid=(10,),
        in_specs=[
            pl.BlockSpec((2, _RB, 128), lambda i: (0, i, 0)),
            pl.BlockSpec((2, _RB, 16), lambda i: (0, i, 0)),
            pl.BlockSpec((2, _RB, 16), lambda i: (0, i, 0)),
            pl.BlockSpec((1, 256), lambda i: (0, 0)),
            pl.BlockSpec((256, 64), lambda i: (0, 0)),
        ],
        out_specs=pl.BlockSpec((_RB, 64), lambda i: (i, 0)),
        out_shape=jax.ShapeDtypeStruct((NP, 64), jnp.float32),
    )(agg1_p, degout_p, degin_p, b1r, W2)


def _k6_body(agg_ref, degi_ref, b2_ref, wp1_ref, bp1_ref, a_ref, b_ref):
    nd = _rsqrt_deg(degi_ref)
    h2 = (agg_ref[0] + agg_ref[1]) * nd + b2_ref[0:1, :]
    a_ref[...] = (jnp.dot(h2, wp1_ref[0:64], preferred_element_type=jnp.float32)
                  + bp1_ref[0:1, :])
    b_ref[...] = jnp.dot(h2, wp1_ref[64:128], preferred_element_type=jnp.float32)


def _k6(agg2_p, degin_p, b2r, Wp1, bp1r):
    return pl.pallas_call(
        _k6_body,
        grid=(10,),
        in_specs=[
            pl.BlockSpec((2, _RB, 64), lambda i: (0, i, 0)),
            pl.BlockSpec((2, _RB, 16), lambda i: (0, i, 0)),
            pl.BlockSpec((1, 64), lambda i: (0, 0)),
            pl.BlockSpec((128, 64), lambda i: (0, 0)),
            pl.BlockSpec((1, 64), lambda i: (0, 0)),
        ],
        out_specs=[
            pl.BlockSpec((_RB, 64), lambda i: (i, 0)),
            pl.BlockSpec((_RB, 64), lambda i: (i, 0)),
        ],
        out_shape=[jax.ShapeDtypeStruct((NP, 64), jnp.float32),
                   jax.ShapeDtypeStruct((NP, 64), jnp.float32)],
    )(agg2_p, degin_p, b2r, Wp1, bp1r)


def _k8_body(c_ref, wp2_ref, bp2_ref, out_ref):
    z = jnp.maximum(c_ref[...], 0.0)
    out_ref[...] = (jnp.sum(z * wp2_ref[0:1, :], axis=1, keepdims=True)
                    + bp2_ref[0:1, :])


def _k8(C, wp2r, bp2r):
    blk = 4096
    return pl.pallas_call(
        _k8_body,
        grid=((2 * EP) // blk,),
        in_specs=[
            pl.BlockSpec((blk, 64), lambda i: (i, 0)),
            pl.BlockSpec((1, 64), lambda i: (0, 0)),
            pl.BlockSpec((1, 1), lambda i: (0, 0)),
        ],
        out_specs=pl.BlockSpec((blk, 1), lambda i: (i, 0)),
        out_shape=jax.ShapeDtypeStruct((2 * EP, 1), jnp.float32),
    )(C, wp2r, bp2r)


def kernel(x, edge_index, pos_edge_index, neg_edge_index,
           W1, b1, W2, b2, Wp1, bp1, Wp2, bp2):
    i32 = jnp.int32
    epad = jnp.full((EP - NEDGE,), NP - 1, i32)
    src_p = jnp.concatenate([edge_index[0], epad]).reshape(ROWS_E, CHUNK)
    dst_p = jnp.concatenate([edge_index[1], epad]).reshape(ROWS_E, CHUNK)
    src_stack = jnp.stack([src_p, src_p + NP])

    zpad = jnp.zeros((EP - NEDGE,), i32)
    u_all = jnp.concatenate([pos_edge_index[0], zpad,
                             neg_edge_index[0], zpad]).reshape(_PRED_ROWS, CHUNK)
    v_all = jnp.concatenate([pos_edge_index[1], zpad,
                             neg_edge_index[1], zpad]).reshape(_PRED_ROWS, CHUNK)

    ones16 = jnp.ones((CHUNK, 16), jnp.float32)
    zeros16 = jnp.zeros((RP, 16), jnp.float32)
    zeros128 = jnp.zeros((RP, 128), jnp.float32)
    zeros64 = jnp.zeros((RP, 64), jnp.float32)

    degout_p, degin_p = _deg_call(src_p, dst_p, ones16, zeros16)

    h1s = _k2(x, W1, degout_p)
    h1s_flat = h1s.reshape(2 * NP, 128)
    agg1_p = _agg1_call(src_stack, dst_p, h1s_flat, zeros128)

    h2in = _k4(agg1_p, degout_p, degin_p, b1.reshape(1, 256), W2)
    agg2_p = _agg2_call(src_p, dst_p, h2in, zeros64)

    A, B = _k6(agg2_p, degin_p, b2.reshape(1, 64), Wp1, bp1.reshape(1, 64))
    C = _pred_call(u_all, v_all, A, B)

    scores = _k8(C, Wp2.reshape(1, 64), bp2.reshape(1, 1))
    pos = scores[:NEDGE, 0]
    neg = scores[EP:EP + NEDGE, 0]
    return (pos, neg)


# speedup vs baseline: 3.6939x; 3.6939x over previous
import functools

import jax
import jax.numpy as jnp
from jax import lax
from jax.experimental import pallas as pl
from jax.experimental.pallas import tpu as pltpu
from jax.experimental.pallas import tpu_sc as plsc

NNODE = 10000
NP = 10240
NEDGE = 160000
EP = 163840
CHUNK = 128
ROWS_E = EP // CHUNK
NC, NS = 2, 16
RP = NP // NS

_MESH = plsc.VectorSubcoreMesh(core_axis_name="c", subcore_axis_name="s")


def _deg_body(src_h, dst_h, ones_h, zeros_h, outp_h, inp_h,
              sidx, didx, ones_v, shout, shin, sem_a, sem_b):
    c = lax.axis_index("c")
    s = lax.axis_index("s")
    nrows = ROWS_E // (NC * NS)
    base = (c * NS + s) * nrows
    pltpu.sync_copy(src_h.at[pl.ds(base, nrows)], sidx)
    pltpu.sync_copy(dst_h.at[pl.ds(base, nrows)], didx)
    pltpu.sync_copy(ones_h, ones_v)
    pltpu.sync_copy(zeros_h, shout.at[pl.ds(s * RP, RP)])
    pltpu.sync_copy(zeros_h, shin.at[pl.ds(s * RP, RP)])
    plsc.subcore_barrier()

    @pl.loop(0, nrows, step=8)
    def _grp(i):
        hs = []
        for k in range(8):
            hs.append(pltpu.async_copy(ones_v, shout.at[sidx.at[i + k]],
                                       sem_a, add=True))
            hs.append(pltpu.async_copy(ones_v, shin.at[didx.at[i + k]],
                                       sem_b, add=True))
        for h in hs:
            h.wait()

    plsc.subcore_barrier()
    pltpu.sync_copy(shout.at[pl.ds(s * RP, RP)], outp_h.at[c, pl.ds(s * RP, RP)])
    pltpu.sync_copy(shin.at[pl.ds(s * RP, RP)], inp_h.at[c, pl.ds(s * RP, RP)])


_deg_call = functools.partial(
    pl.kernel,
    out_type=[jax.ShapeDtypeStruct((NC, NP, 16), jnp.float32),
              jax.ShapeDtypeStruct((NC, NP, 16), jnp.float32)],
    mesh=_MESH,
    compiler_params=pltpu.CompilerParams(use_tc_tiling_on_sc=False),
    scratch_types=[
        pltpu.VMEM((ROWS_E // (NC * NS), CHUNK), jnp.int32),
        pltpu.VMEM((ROWS_E // (NC * NS), CHUNK), jnp.int32),
        pltpu.VMEM((CHUNK, 16), jnp.float32),
        pltpu.VMEM_SHARED((NP, 16), jnp.float32),
        pltpu.VMEM_SHARED((NP, 16), jnp.float32),
        pltpu.SemaphoreType.DMA,
        pltpu.SemaphoreType.DMA,
    ],
)(_deg_body)


def _make_agg(nfeat, per_sc_edges_split, nbuf, phases):
    if per_sc_edges_split:
        nrows = ROWS_E // (NC * NS)
    else:
        nrows = ROWS_E // NS
    rpp = nrows // phases

    def body(src_h, dst_h, h_h, zeros_h, agg_h,
             sidx, didx, *rest):
        bufs = rest[:nbuf]
        shared = rest[nbuf]
        gsem = rest[nbuf + 1:2 * nbuf + 1]
        ssem = rest[2 * nbuf + 1:3 * nbuf + 1]
        c = lax.axis_index("c")
        s = lax.axis_index("s")
        if per_sc_edges_split:
            base = (c * NS + s) * nrows
        else:
            base = s * nrows
        pltpu.sync_copy(zeros_h, shared.at[pl.ds(s * RP, RP)])
        plsc.subcore_barrier()

        for p in range(phases):
            if per_sc_edges_split:
                pltpu.sync_copy(src_h.at[pl.ds(base + p * rpp, rpp)], sidx)
            else:
                pltpu.sync_copy(src_h.at[c, pl.ds(base + p * rpp, rpp)], sidx)
            pltpu.sync_copy(dst_h.at[pl.ds(base + p * rpp, rpp)], didx)

            @pl.loop(0, rpp, step=nbuf)
            def _grp(i):
                hg = [pltpu.async_copy(h_h.at[sidx.at[i + k]], bufs[k],
                                       gsem[k]) for k in range(nbuf)]
                hs = []
                for k in range(nbuf):
                    hg[k].wait()
                    hs.append(pltpu.async_copy(bufs[k],
                                               shared.at[didx.at[i + k]],
                                               ssem[k], add=True))
                for h in hs:
                    h.wait()

        plsc.subcore_barrier()
        pltpu.sync_copy(shared.at[pl.ds(s * RP, RP)],
                        agg_h.at[c, pl.ds(s * RP, RP)])

    return functools.partial(
        pl.kernel,
        out_type=jax.ShapeDtypeStruct((NC, NP, nfeat), jnp.float32),
        mesh=_MESH,
        compiler_params=pltpu.CompilerParams(use_tc_tiling_on_sc=False),
        scratch_types=(
            [pltpu.VMEM((rpp, CHUNK), jnp.int32),
             pltpu.VMEM((rpp, CHUNK), jnp.int32)]
            + [pltpu.VMEM((CHUNK, nfeat), jnp.float32)] * nbuf
            + [pltpu.VMEM_SHARED((NP, nfeat), jnp.float32)]
            + [pltpu.SemaphoreType.DMA] * (2 * nbuf)
        ),
    )(body)


_agg1_call = _make_agg(128, per_sc_edges_split=False, nbuf=2, phases=2)
_agg2_call = _make_agg(64, per_sc_edges_split=True, nbuf=4, phases=1)


_PRED_ROWS = (2 * EP) // CHUNK


def _pred_body(u_h, v_h, a_h, b_h, c_h,
               uidx, vidx, b0, b1, b2, b3,
               sa0, sa1, sa2, sa3, sb0, sb1, sb2, sb3,
               so0, so1, so2, so3):
    c = lax.axis_index("c")
    s = lax.axis_index("s")
    nrows = _PRED_ROWS // (NC * NS)
    nbuf = 4
    bufs = (b0, b1, b2, b3)
    asem = (sa0, sa1, sa2, sa3)
    bsem = (sb0, sb1, sb2, sb3)
    osem = (so0, so1, so2, so3)
    base = (c * NS + s) * nrows
    pltpu.sync_copy(u_h.at[pl.ds(base, nrows)], uidx)
    pltpu.sync_copy(v_h.at[pl.ds(base, nrows)], vidx)

    @pl.loop(0, nrows, step=nbuf)
    def _grp(i):
        ha = [pltpu.async_copy(a_h.at[uidx.at[i + k]], bufs[k], asem[k])
              for k in range(nbuf)]
        hb = []
        for k in range(nbuf):
            ha[k].wait()
            hb.append(pltpu.async_copy(b_h.at[vidx.at[i + k]], bufs[k],
                                       bsem[k], add=True))
        ho = []
        for k in range(nbuf):
            hb[k].wait()
            row0 = (base + i + k) * CHUNK
            ho.append(pltpu.async_copy(bufs[k], c_h.at[pl.ds(row0, CHUNK)],
                                       osem[k]))
        for h in ho:
            h.wait()


_pred_call = functools.partial(
    pl.kernel,
    out_type=jax.ShapeDtypeStruct((2 * EP, 64), jnp.float32),
    mesh=_MESH,
    compiler_params=pltpu.CompilerParams(use_tc_tiling_on_sc=False),
    scratch_types=[
        pltpu.VMEM((_PRED_ROWS // (NC * NS), CHUNK), jnp.int32),
        pltpu.VMEM((_PRED_ROWS // (NC * NS), CHUNK), jnp.int32),
        pltpu.VMEM((CHUNK, 64), jnp.float32),
        pltpu.VMEM((CHUNK, 64), jnp.float32),
        pltpu.VMEM((CHUNK, 64), jnp.float32),
        pltpu.VMEM((CHUNK, 64), jnp.float32),
        pltpu.SemaphoreType.DMA, pltpu.SemaphoreType.DMA,
        pltpu.SemaphoreType.DMA, pltpu.SemaphoreType.DMA,
        pltpu.SemaphoreType.DMA, pltpu.SemaphoreType.DMA,
        pltpu.SemaphoreType.DMA, pltpu.SemaphoreType.DMA,
        pltpu.SemaphoreType.DMA, pltpu.SemaphoreType.DMA,
        pltpu.SemaphoreType.DMA, pltpu.SemaphoreType.DMA,
    ],
)(_pred_body)


_RB = 1000


def _rsqrt_deg(ref):
    d = ref[0, :, 0:1] + ref[1, :, 0:1]
    return lax.rsqrt(jnp.maximum(d, 1.0))


def _k2_body(x_ref, w_ref, dego_ref, out_ref):
    ns = _rsqrt_deg(dego_ref)
    xw = jnp.dot(x_ref[...], w_ref[...], preferred_element_type=jnp.float32)
    out_ref[0] = xw * ns


def _k2(x, W1, degout_p):
    return pl.pallas_call(
        _k2_body,
        grid=(2, 10),
        in_specs=[
            pl.BlockSpec((_RB, 512), lambda h, i: (i, 0)),
            pl.BlockSpec((512, 128), lambda h, i: (0, h)),
            pl.BlockSpec((2, _RB, 16), lambda h, i: (0, i, 0)),
        ],
        out_specs=pl.BlockSpec((1, _RB, 128), lambda h, i: (h, i, 0)),
        out_shape=jax.ShapeDtypeStruct((2, NP, 128), jnp.float32),
    )(x, W1, degout_p)


def _k4_body(agg_ref, dego_ref, degi_ref, b1_ref, w2_ref, out_ref):
    ns = _rsqrt_deg(dego_ref)
    nd = _rsqrt_deg(degi_ref)
    a = jnp.maximum(agg_ref[0] * nd + b1_ref[0:1, 0:128], 0.0)
    b = jnp.maximum(agg_ref[1] * nd + b1_ref[0:1, 128:256], 0.0)
    t = (jnp.dot(a, w2_ref[0:128], preferred_element_type=jnp.float32)
         + jnp.dot(b, w2_ref[128:256], preferred_element_type=jnp.float32))
    out_ref[...] = t * ns


def _k4(agg1_p, degout_p, degin_p, b1r, W2):
    return pl.pallas_call(
        _k4_body,
        grid=(10,),
        in_specs=[
            pl.BlockSpec((2, _RB, 128), lambda i: (0, i, 0)),
            pl.BlockSpec((2, _RB, 16), lambda i: (0, i, 0)),
            pl.BlockSpec((2, _RB, 16), lambda i: (0, i, 0)),
            pl.BlockSpec((1, 256), lambda i: (0, 0)),
            pl.BlockSpec((256, 64), lambda i: (0, 0)),
        ],
        out_specs=pl.BlockSpec((_RB, 64), lambda i: (i, 0)),
        out_shape=jax.ShapeDtypeStruct((NP, 64), jnp.float32),
    )(agg1_p, degout_p, degin_p, b1r, W2)


def _k6_body(agg_ref, degi_ref, b2_ref, wp1_ref, bp1_ref, a_ref, b_ref):
    nd = _rsqrt_deg(degi_ref)
    h2 = (agg_ref[0] + agg_ref[1]) * nd + b2_ref[0:1, :]
    a_ref[...] = (jnp.dot(h2, wp1_ref[0:64], preferred_element_type=jnp.float32)
                  + bp1_ref[0:1, :])
    b_ref[...] = jnp.dot(h2, wp1_ref[64:128], preferred_element_type=jnp.float32)


def _k6(agg2_p, degin_p, b2r, Wp1, bp1r):
    return pl.pallas_call(
        _k6_body,
        grid=(10,),
        in_specs=[
            pl.BlockSpec((2, _RB, 64), lambda i: (0, i, 0)),
            pl.BlockSpec((2, _RB, 16), lambda i: (0, i, 0)),
            pl.BlockSpec((1, 64), lambda i: (0, 0)),
            pl.BlockSpec((128, 64), lambda i: (0, 0)),
            pl.BlockSpec((1, 64), lambda i: (0, 0)),
        ],
        out_specs=[
            pl.BlockSpec((_RB, 64), lambda i: (i, 0)),
            pl.BlockSpec((_RB, 64), lambda i: (i, 0)),
        ],
        out_shape=[jax.ShapeDtypeStruct((NP, 64), jnp.float32),
                   jax.ShapeDtypeStruct((NP, 64), jnp.float32)],
    )(agg2_p, degin_p, b2r, Wp1, bp1r)


def _k8_body(c_ref, wp2_ref, bp2_ref, out_ref):
    z = jnp.maximum(c_ref[...], 0.0)
    out_ref[...] = (jnp.sum(z * wp2_ref[0:1, :], axis=1, keepdims=True)
                    + bp2_ref[0:1, :])


def _k8(C, wp2r, bp2r):
    blk = 4096
    return pl.pallas_call(
        _k8_body,
        grid=((2 * EP) // blk,),
        in_specs=[
            pl.BlockSpec((blk, 64), lambda i: (i, 0)),
            pl.BlockSpec((1, 64), lambda i: (0, 0)),
            pl.BlockSpec((1, 1), lambda i: (0, 0)),
        ],
        out_specs=pl.BlockSpec((blk, 1), lambda i: (i, 0)),
        out_shape=jax.ShapeDtypeStruct((2 * EP, 1), jnp.float32),
    )(C, wp2r, bp2r)


def kernel(x, edge_index, pos_edge_index, neg_edge_index,
           W1, b1, W2, b2, Wp1, bp1, Wp2, bp2):
    i32 = jnp.int32
    epad = jnp.full((EP - NEDGE,), NP - 1, i32)
    src_p = jnp.concatenate([edge_index[0], epad]).reshape(ROWS_E, CHUNK)
    dst_p = jnp.concatenate([edge_index[1], epad]).reshape(ROWS_E, CHUNK)
    src_stack = jnp.stack([src_p, src_p + NP])

    zpad = jnp.zeros((EP - NEDGE,), i32)
    u_all = jnp.concatenate([pos_edge_index[0], zpad,
                             neg_edge_index[0], zpad]).reshape(_PRED_ROWS, CHUNK)
    v_all = jnp.concatenate([pos_edge_index[1], zpad,
                             neg_edge_index[1], zpad]).reshape(_PRED_ROWS, CHUNK)

    ones16 = jnp.ones((CHUNK, 16), jnp.float32)
    zeros16 = jnp.zeros((RP, 16), jnp.float32)
    zeros128 = jnp.zeros((RP, 128), jnp.float32)
    zeros64 = jnp.zeros((RP, 64), jnp.float32)

    degout_p, degin_p = _deg_call(src_p, dst_p, ones16, zeros16)

    h1s = _k2(x, W1, degout_p)
    h1s_flat = h1s.reshape(2 * NP, 128)
    agg1_p = _agg1_call(src_stack, dst_p, h1s_flat, zeros128)

    h2in = _k4(agg1_p, degout_p, degin_p, b1.reshape(1, 256), W2)
    agg2_p = _agg2_call(src_p, dst_p, h2in, zeros64)

    A, B = _k6(agg2_p, degin_p, b2.reshape(1, 64), Wp1, bp1.reshape(1, 64))
    C = _pred_call(u_all, v_all, A, B)

    scores = _k8(C, Wp2.reshape(1, 64), bp2.reshape(1, 1))
    pos = scores[:NEDGE, 0]
    neg = scores[EP:EP + NEDGE, 0]
    return (pos, neg)


# speedup vs baseline: 3.7935x; 1.0270x over previous
import functools

import jax
import jax.numpy as jnp
from jax import lax
from jax.experimental import pallas as pl
from jax.experimental.pallas import tpu as pltpu
from jax.experimental.pallas import tpu_sc as plsc

NNODE = 10000
NP = 10240
NEDGE = 160000
EP = 163840
CHUNK = 128
ROWS_E = EP // CHUNK
NC, NS = 2, 16
RP = NP // NS

_MESH = plsc.VectorSubcoreMesh(core_axis_name="c", subcore_axis_name="s")


def _deg_body(src_h, dst_h, ones_h, zeros_h, outp_h, inp_h,
              sidx, didx, ones_v, shout, shin, sem_a, sem_b):
    c = lax.axis_index("c")
    s = lax.axis_index("s")
    nrows = ROWS_E // (NC * NS)
    base = (c * NS + s) * nrows
    pltpu.sync_copy(src_h.at[pl.ds(base, nrows)], sidx)
    pltpu.sync_copy(dst_h.at[pl.ds(base, nrows)], didx)
    pltpu.sync_copy(ones_h, ones_v)
    pltpu.sync_copy(zeros_h, shout.at[pl.ds(s * RP, RP)])
    pltpu.sync_copy(zeros_h, shin.at[pl.ds(s * RP, RP)])
    plsc.subcore_barrier()

    @pl.loop(0, nrows, step=8)
    def _grp(i):
        hs = []
        for k in range(8):
            hs.append(pltpu.async_copy(ones_v, shout.at[sidx.at[i + k]],
                                       sem_a, add=True))
            hs.append(pltpu.async_copy(ones_v, shin.at[didx.at[i + k]],
                                       sem_b, add=True))
        for h in hs:
            h.wait()

    plsc.subcore_barrier()
    pltpu.sync_copy(shout.at[pl.ds(s * RP, RP)], outp_h.at[c, pl.ds(s * RP, RP)])
    pltpu.sync_copy(shin.at[pl.ds(s * RP, RP)], inp_h.at[c, pl.ds(s * RP, RP)])


_deg_call = functools.partial(
    pl.kernel,
    out_type=[jax.ShapeDtypeStruct((NC, NP, 16), jnp.float32),
              jax.ShapeDtypeStruct((NC, NP, 16), jnp.float32)],
    mesh=_MESH,
    compiler_params=pltpu.CompilerParams(use_tc_tiling_on_sc=False),
    scratch_types=[
        pltpu.VMEM((ROWS_E // (NC * NS), CHUNK), jnp.int32),
        pltpu.VMEM((ROWS_E // (NC * NS), CHUNK), jnp.int32),
        pltpu.VMEM((CHUNK, 16), jnp.float32),
        pltpu.VMEM_SHARED((NP, 16), jnp.float32),
        pltpu.VMEM_SHARED((NP, 16), jnp.float32),
        pltpu.SemaphoreType.DMA,
        pltpu.SemaphoreType.DMA,
    ],
)(_deg_body)


def _make_agg(nfeat, per_sc_edges_split, nbuf, phases, chunk=CHUNK):
    rows_e = EP // chunk
    if per_sc_edges_split:
        nrows = rows_e // (NC * NS)
    else:
        nrows = rows_e // NS
    rpp = nrows // phases

    def body(src_h, dst_h, h_h, zeros_h, agg_h,
             sidx, didx, *rest):
        bufs = rest[:nbuf]
        shared = rest[nbuf]
        gsem = rest[nbuf + 1:2 * nbuf + 1]
        ssem = rest[2 * nbuf + 1:3 * nbuf + 1]
        c = lax.axis_index("c")
        s = lax.axis_index("s")
        if per_sc_edges_split:
            base = (c * NS + s) * nrows
        else:
            base = s * nrows
        pltpu.sync_copy(zeros_h, shared.at[pl.ds(s * RP, RP)])
        plsc.subcore_barrier()

        for p in range(phases):
            if per_sc_edges_split:
                pltpu.sync_copy(src_h.at[pl.ds(base + p * rpp, rpp)], sidx)
            else:
                pltpu.sync_copy(src_h.at[c, pl.ds(base + p * rpp, rpp)], sidx)
            pltpu.sync_copy(dst_h.at[pl.ds(base + p * rpp, rpp)], didx)

            @pl.loop(0, rpp, step=nbuf)
            def _grp(i):
                hg = [pltpu.async_copy(h_h.at[sidx.at[i + k]], bufs[k],
                                       gsem[k]) for k in range(nbuf)]
                hs = []
                for k in range(nbuf):
                    hg[k].wait()
                    hs.append(pltpu.async_copy(bufs[k],
                                               shared.at[didx.at[i + k]],
                                               ssem[k], add=True))
                for h in hs:
                    h.wait()

        plsc.subcore_barrier()
        pltpu.sync_copy(shared.at[pl.ds(s * RP, RP)],
                        agg_h.at[c, pl.ds(s * RP, RP)])

    return functools.partial(
        pl.kernel,
        out_type=jax.ShapeDtypeStruct((NC, NP, nfeat), jnp.float32),
        mesh=_MESH,
        compiler_params=pltpu.CompilerParams(use_tc_tiling_on_sc=False),
        scratch_types=(
            [pltpu.VMEM((rpp, chunk), jnp.int32),
             pltpu.VMEM((rpp, chunk), jnp.int32)]
            + [pltpu.VMEM((chunk, nfeat), jnp.float32)] * nbuf
            + [pltpu.VMEM_SHARED((NP, nfeat), jnp.float32)]
            + [pltpu.SemaphoreType.DMA] * (2 * nbuf)
        ),
    )(body)


_agg1_call = _make_agg(128, per_sc_edges_split=False, nbuf=4, phases=2,
                       chunk=64)
_agg2_call = _make_agg(64, per_sc_edges_split=True, nbuf=8, phases=1)


_PRED_ROWS = (2 * EP) // CHUNK


_PRED_NBUF = 8


def _pred_body(u_h, v_h, a_h, b_h, c_h, uidx, vidx, *rest):
    c = lax.axis_index("c")
    s = lax.axis_index("s")
    nrows = _PRED_ROWS // (NC * NS)
    nbuf = _PRED_NBUF
    bufs = rest[:nbuf]
    asem = rest[nbuf:2 * nbuf]
    bsem = rest[2 * nbuf:3 * nbuf]
    osem = rest[3 * nbuf:4 * nbuf]
    base = (c * NS + s) * nrows
    pltpu.sync_copy(u_h.at[pl.ds(base, nrows)], uidx)
    pltpu.sync_copy(v_h.at[pl.ds(base, nrows)], vidx)

    @pl.loop(0, nrows, step=nbuf)
    def _grp(i):
        ha = [pltpu.async_copy(a_h.at[uidx.at[i + k]], bufs[k], asem[k])
              for k in range(nbuf)]
        hb = []
        for k in range(nbuf):
            ha[k].wait()
            hb.append(pltpu.async_copy(b_h.at[vidx.at[i + k]], bufs[k],
                                       bsem[k], add=True))
        ho = []
        for k in range(nbuf):
            hb[k].wait()
            row0 = (base + i + k) * CHUNK
            ho.append(pltpu.async_copy(bufs[k], c_h.at[pl.ds(row0, CHUNK)],
                                       osem[k]))
        for h in ho:
            h.wait()


_pred_call = functools.partial(
    pl.kernel,
    out_type=jax.ShapeDtypeStruct((2 * EP, 64), jnp.float32),
    mesh=_MESH,
    compiler_params=pltpu.CompilerParams(use_tc_tiling_on_sc=False),
    scratch_types=(
        [pltpu.VMEM((_PRED_ROWS // (NC * NS), CHUNK), jnp.int32),
         pltpu.VMEM((_PRED_ROWS // (NC * NS), CHUNK), jnp.int32)]
        + [pltpu.VMEM((CHUNK, 64), jnp.float32)] * _PRED_NBUF
        + [pltpu.SemaphoreType.DMA] * (3 * _PRED_NBUF)
    ),
)(_pred_body)


_RB = 1000


def _rsqrt_deg(ref):
    d = ref[0, :, 0:1] + ref[1, :, 0:1]
    return lax.rsqrt(jnp.maximum(d, 1.0))


def _k2_body(x_ref, w_ref, dego_ref, out_ref):
    ns = _rsqrt_deg(dego_ref)
    xw = jnp.dot(x_ref[...], w_ref[...], preferred_element_type=jnp.float32)
    out_ref[0] = xw * ns


def _k2(x, W1, degout_p):
    return pl.pallas_call(
        _k2_body,
        grid=(2, 10),
        in_specs=[
            pl.BlockSpec((_RB, 512), lambda h, i: (i, 0)),
            pl.BlockSpec((512, 128), lambda h, i: (0, h)),
            pl.BlockSpec((2, _RB, 16), lambda h, i: (0, i, 0)),
        ],
        out_specs=pl.BlockSpec((1, _RB, 128), lambda h, i: (h, i, 0)),
        out_shape=jax.ShapeDtypeStruct((2, NP, 128), jnp.float32),
    )(x, W1, degout_p)


def _k4_body(agg_ref, dego_ref, degi_ref, b1_ref, w2_ref, out_ref):
    ns = _rsqrt_deg(dego_ref)
    nd = _rsqrt_deg(degi_ref)
    a = jnp.maximum(agg_ref[0] * nd + b1_ref[0:1, 0:128], 0.0)
    b = jnp.maximum(agg_ref[1] * nd + b1_ref[0:1, 128:256], 0.0)
    t = (jnp.dot(a, w2_ref[0:128], preferred_element_type=jnp.float32)
         + jnp.dot(b, w2_ref[128:256], preferred_element_type=jnp.float32))
    out_ref[...] = t * ns


def _k4(agg1_p, degout_p, degin_p, b1r, W2):
    return pl.pallas_call(
        _k4_body,
        grid=(10,),
        in_specs=[
            pl.BlockSpec((2, _RB, 128), lambda i: (0, i, 0)),
            pl.BlockSpec((2, _RB, 16), lambda i: (0, i, 0)),
            pl.BlockSpec((2, _RB, 16), lambda i: (0, i, 0)),
            pl.BlockSpec((1, 256), lambda i: (0, 0)),
            pl.BlockSpec((256, 64), lambda i: (0, 0)),
        ],
        out_specs=pl.BlockSpec((_RB, 64), lambda i: (i, 0)),
        out_shape=jax.ShapeDtypeStruct((NP, 64), jnp.float32),
    )(agg1_p, degout_p, degin_p, b1r, W2)


def _k6_body(agg_ref, degi_ref, b2_ref, wp1_ref, bp1_ref, a_ref, b_ref):
    nd = _rsqrt_deg(degi_ref)
    h2 = (agg_ref[0] + agg_ref[1]) * nd + b2_ref[0:1, :]
    a_ref[...] = (jnp.dot(h2, wp1_ref[0:64], preferred_element_type=jnp.float32)
                  + bp1_ref[0:1, :])
    b_ref[...] = jnp.dot(h2, wp1_ref[64:128], preferred_element_type=jnp.float32)


def _k6(agg2_p, degin_p, b2r, Wp1, bp1r):
    return pl.pallas_call(
        _k6_body,
        grid=(10,),
        in_specs=[
            pl.BlockSpec((2, _RB, 64), lambda i: (0, i, 0)),
            pl.BlockSpec((2, _RB, 16), lambda i: (0, i, 0)),
            pl.BlockSpec((1, 64), lambda i: (0, 0)),
            pl.BlockSpec((128, 64), lambda i: (0, 0)),
            pl.BlockSpec((1, 64), lambda i: (0, 0)),
        ],
        out_specs=[
            pl.BlockSpec((_RB, 64), lambda i: (i, 0)),
            pl.BlockSpec((_RB, 64), lambda i: (i, 0)),
        ],
        out_shape=[jax.ShapeDtypeStruct((NP, 64), jnp.float32),
                   jax.ShapeDtypeStruct((NP, 64), jnp.float32)],
    )(agg2_p, degin_p, b2r, Wp1, bp1r)


def _k8_body(c_ref, wp2_ref, bp2_ref, out_ref):
    z = jnp.maximum(c_ref[...], 0.0)
    out_ref[...] = (jnp.sum(z * wp2_ref[0:1, :], axis=1, keepdims=True)
                    + bp2_ref[0:1, :])


def _k8(C, wp2r, bp2r):
    blk = 4096
    return pl.pallas_call(
        _k8_body,
        grid=((2 * EP) // blk,),
        in_specs=[
            pl.BlockSpec((blk, 64), lambda i: (i, 0)),
            pl.BlockSpec((1, 64), lambda i: (0, 0)),
            pl.BlockSpec((1, 1), lambda i: (0, 0)),
        ],
        out_specs=pl.BlockSpec((blk, 1), lambda i: (i, 0)),
        out_shape=jax.ShapeDtypeStruct((2 * EP, 1), jnp.float32),
    )(C, wp2r, bp2r)


def kernel(x, edge_index, pos_edge_index, neg_edge_index,
           W1, b1, W2, b2, Wp1, bp1, Wp2, bp2):
    i32 = jnp.int32
    epad = jnp.full((EP - NEDGE,), NP - 1, i32)
    src_p = jnp.concatenate([edge_index[0], epad]).reshape(ROWS_E, CHUNK)
    dst_p = jnp.concatenate([edge_index[1], epad]).reshape(ROWS_E, CHUNK)
    src_stack = jnp.stack([src_p, src_p + NP])

    zpad = jnp.zeros((EP - NEDGE,), i32)
    u_all = jnp.concatenate([pos_edge_index[0], zpad,
                             neg_edge_index[0], zpad]).reshape(_PRED_ROWS, CHUNK)
    v_all = jnp.concatenate([pos_edge_index[1], zpad,
                             neg_edge_index[1], zpad]).reshape(_PRED_ROWS, CHUNK)

    ones16 = jnp.ones((CHUNK, 16), jnp.float32)
    zeros16 = jnp.zeros((RP, 16), jnp.float32)
    zeros128 = jnp.zeros((RP, 128), jnp.float32)
    zeros64 = jnp.zeros((RP, 64), jnp.float32)

    degout_p, degin_p = _deg_call(src_p, dst_p, ones16, zeros16)

    h1s = _k2(x, W1, degout_p)
    h1s_flat = h1s.reshape(2 * NP, 128)
    agg1_p = _agg1_call(src_stack.reshape(2, EP // 64, 64),
                        dst_p.reshape(EP // 64, 64), h1s_flat, zeros128)

    h2in = _k4(agg1_p, degout_p, degin_p, b1.reshape(1, 256), W2)
    agg2_p = _agg2_call(src_p, dst_p, h2in, zeros64)

    A, B = _k6(agg2_p, degin_p, b2.reshape(1, 64), Wp1, bp1.reshape(1, 64))
    C = _pred_call(u_all, v_all, A, B)

    scores = _k8(C, Wp2.reshape(1, 64), bp2.reshape(1, 1))
    pos = scores[:NEDGE, 0]
    neg = scores[EP:EP + NEDGE, 0]
    return (pos, neg)


# speedup vs baseline: 5.7910x; 1.5266x over previous
import functools

import jax
import jax.numpy as jnp
from jax import lax
from jax.experimental import pallas as pl
from jax.experimental.pallas import tpu as pltpu
from jax.experimental.pallas import tpu_sc as plsc

NNODE = 10000
NP = 10240
NEDGE = 160000
EP = 163840
CHUNK = 128
ROWS_E = EP // CHUNK
NC, NS = 2, 16
RP = NP // NS

_MESH = plsc.VectorSubcoreMesh(core_axis_name="c", subcore_axis_name="s")


def _deg_body(src_h, dst_h, ones_h, zeros_h, outp_h, inp_h,
              sidx, didx, ones_v, shout, shin, sem_a, sem_b):
    c = lax.axis_index("c")
    s = lax.axis_index("s")
    nrows = ROWS_E // (NC * NS)
    base = (c * NS + s) * nrows
    pltpu.sync_copy(src_h.at[pl.ds(base, nrows)], sidx)
    pltpu.sync_copy(dst_h.at[pl.ds(base, nrows)], didx)
    pltpu.sync_copy(ones_h, ones_v)
    pltpu.sync_copy(zeros_h, shout.at[pl.ds(s * RP, RP)])
    pltpu.sync_copy(zeros_h, shin.at[pl.ds(s * RP, RP)])
    plsc.subcore_barrier()

    @pl.loop(0, nrows, step=8)
    def _grp(i):
        hs = []
        for k in range(8):
            hs.append(pltpu.async_copy(ones_v, shout.at[sidx.at[i + k]],
                                       sem_a, add=True))
            hs.append(pltpu.async_copy(ones_v, shin.at[didx.at[i + k]],
                                       sem_b, add=True))
        for h in hs:
            h.wait()

    plsc.subcore_barrier()
    pltpu.sync_copy(shout.at[pl.ds(s * RP, RP)], outp_h.at[c, pl.ds(s * RP, RP)])
    pltpu.sync_copy(shin.at[pl.ds(s * RP, RP)], inp_h.at[c, pl.ds(s * RP, RP)])


_deg_call = functools.partial(
    pl.kernel,
    out_type=[jax.ShapeDtypeStruct((NC, NP, 16), jnp.float32),
              jax.ShapeDtypeStruct((NC, NP, 16), jnp.float32)],
    mesh=_MESH,
    compiler_params=pltpu.CompilerParams(use_tc_tiling_on_sc=False),
    scratch_types=[
        pltpu.VMEM((ROWS_E // (NC * NS), CHUNK), jnp.int32),
        pltpu.VMEM((ROWS_E // (NC * NS), CHUNK), jnp.int32),
        pltpu.VMEM((CHUNK, 16), jnp.float32),
        pltpu.VMEM_SHARED((NP, 16), jnp.float32),
        pltpu.VMEM_SHARED((NP, 16), jnp.float32),
        pltpu.SemaphoreType.DMA,
        pltpu.SemaphoreType.DMA,
    ],
)(_deg_body)


def _make_agg(passes, per_sc_edges_split, nbuf, phases, nquarter, chunk=64):
    rows_e = EP // chunk
    if per_sc_edges_split:
        nrows = rows_e // (NC * NS)
    else:
        nrows = rows_e // NS
    rpp = nrows // phases
    nout = NC if per_sc_edges_split else passes * NC

    def body(src_h, dst_h, h_h, zeros_h, agg_h, sidx, didx, *rest):
        bufs = rest[:nbuf]
        table = rest[nbuf]
        acc = rest[nbuf + 1]
        gsem = rest[nbuf + 2:2 * nbuf + 2]
        ssem = rest[2 * nbuf + 2:3 * nbuf + 2]
        c = lax.axis_index("c")
        s = lax.axis_index("s")
        if per_sc_edges_split:
            base = (c * NS + s) * nrows
        else:
            base = s * nrows
        rows = pl.ds(s * RP, RP)

        for q in range(passes):
            qidx = c * passes + q if not per_sc_edges_split else 0
            pltpu.sync_copy(h_h.at[qidx, rows], table.at[rows])
            pltpu.sync_copy(zeros_h, acc.at[rows])
            plsc.subcore_barrier()

            for p in range(phases):
                if per_sc_edges_split:
                    pltpu.sync_copy(src_h.at[pl.ds(base + p * rpp, rpp)], sidx)
                else:
                    pltpu.sync_copy(src_h.at[pl.ds(base + p * rpp, rpp)], sidx)
                pltpu.sync_copy(dst_h.at[pl.ds(base + p * rpp, rpp)], didx)

                @pl.loop(0, rpp, step=nbuf)
                def _grp(i):
                    hg = [pltpu.async_copy(table.at[sidx.at[i + k]], bufs[k],
                                           gsem[k]) for k in range(nbuf)]
                    hs = []
                    for k in range(nbuf):
                        hg[k].wait()
                        hs.append(pltpu.async_copy(bufs[k],
                                                   acc.at[didx.at[i + k]],
                                                   ssem[k], add=True))
                    for h in hs:
                        h.wait()

            plsc.subcore_barrier()
            out_idx = c if per_sc_edges_split else c * passes + q
            pltpu.sync_copy(acc.at[rows], agg_h.at[out_idx, rows])

    return functools.partial(
        pl.kernel,
        out_type=jax.ShapeDtypeStruct((nout, NP, 64), jnp.float32),
        mesh=_MESH,
        compiler_params=pltpu.CompilerParams(use_tc_tiling_on_sc=False),
        scratch_types=(
            [pltpu.VMEM((rpp, chunk), jnp.int32),
             pltpu.VMEM((rpp, chunk), jnp.int32)]
            + [pltpu.VMEM((chunk, 64), jnp.float32)] * nbuf
            + [pltpu.VMEM_SHARED((NP, 64), jnp.float32),
               pltpu.VMEM_SHARED((NP, 64), jnp.float32)]
            + [pltpu.SemaphoreType.DMA] * (2 * nbuf)
        ),
    )(body)


_agg1_call = _make_agg(passes=2, per_sc_edges_split=False, nbuf=4, phases=2,
                       nquarter=4)
_agg2_call = _make_agg(passes=1, per_sc_edges_split=True, nbuf=4, phases=1,
                       nquarter=1)


_PRED_ROWS = (2 * EP) // CHUNK


_PRED_NBUF = 8
_PRED_CHUNK = 64
_PRED_TROWS = (2 * EP) // _PRED_CHUNK // (NC * NS)
_PRED_PH = 2


def _pred_body(u_h, v_h, a_h, b_h, c_h, uidx, vidx, *rest):
    c = lax.axis_index("c")
    s = lax.axis_index("s")
    nbuf = _PRED_NBUF
    rpp = _PRED_TROWS // _PRED_PH
    bufs = rest[:nbuf]
    sha = rest[nbuf]
    shb = rest[nbuf + 1]
    asem = rest[nbuf + 2:2 * nbuf + 2]
    bsem = rest[2 * nbuf + 2:3 * nbuf + 2]
    osem = rest[3 * nbuf + 2:4 * nbuf + 2]
    base = (c * NS + s) * _PRED_TROWS
    rows = pl.ds(s * RP, RP)
    pltpu.sync_copy(a_h.at[rows], sha.at[rows])
    pltpu.sync_copy(b_h.at[rows], shb.at[rows])
    plsc.subcore_barrier()

    for p in range(_PRED_PH):
        pltpu.sync_copy(u_h.at[pl.ds(base + p * rpp, rpp)], uidx)
        pltpu.sync_copy(v_h.at[pl.ds(base + p * rpp, rpp)], vidx)

        @pl.loop(0, rpp, step=nbuf)
        def _grp(i):
            ha = [pltpu.async_copy(sha.at[uidx.at[i + k]], bufs[k], asem[k])
                  for k in range(nbuf)]
            hb = []
            for k in range(nbuf):
                ha[k].wait()
                hb.append(pltpu.async_copy(shb.at[vidx.at[i + k]], bufs[k],
                                           bsem[k], add=True))
            ho = []
            for k in range(nbuf):
                hb[k].wait()
                row0 = (base + p * rpp + i + k) * _PRED_CHUNK
                ho.append(pltpu.async_copy(bufs[k],
                                           c_h.at[pl.ds(row0, _PRED_CHUNK)],
                                           osem[k]))
            for h in ho:
                h.wait()


_pred_call = functools.partial(
    pl.kernel,
    out_type=jax.ShapeDtypeStruct((2 * EP, 64), jnp.float32),
    mesh=_MESH,
    compiler_params=pltpu.CompilerParams(use_tc_tiling_on_sc=False),
    scratch_types=(
        [pltpu.VMEM((_PRED_TROWS // _PRED_PH, _PRED_CHUNK), jnp.int32),
         pltpu.VMEM((_PRED_TROWS // _PRED_PH, _PRED_CHUNK), jnp.int32)]
        + [pltpu.VMEM((_PRED_CHUNK, 64), jnp.float32)] * _PRED_NBUF
        + [pltpu.VMEM_SHARED((NP, 64), jnp.float32),
           pltpu.VMEM_SHARED((NP, 64), jnp.float32)]
        + [pltpu.SemaphoreType.DMA] * (3 * _PRED_NBUF)
    ),
)(_pred_body)


_RB = 1000


def _rsqrt_deg(ref):
    d = ref[0, :, 0:1] + ref[1, :, 0:1]
    return lax.rsqrt(jnp.maximum(d, 1.0))


def _k2_body(x_ref, w_ref, dego_ref, out_ref):
    ns = _rsqrt_deg(dego_ref)
    xw = jnp.dot(x_ref[...], w_ref[0], preferred_element_type=jnp.float32)
    out_ref[0] = xw * ns


def _k2(x, W1, degout_p):
    return pl.pallas_call(
        _k2_body,
        grid=(10, 4),
        in_specs=[
            pl.BlockSpec((_RB, 512), lambda i, h: (i, 0)),
            pl.BlockSpec((1, 512, 64), lambda i, h: (h, 0, 0)),
            pl.BlockSpec((2, _RB, 16), lambda i, h: (0, i, 0)),
        ],
        out_specs=pl.BlockSpec((1, _RB, 64), lambda i, h: (h, i, 0)),
        out_shape=jax.ShapeDtypeStruct((4, NP, 64), jnp.float32),
    )(x, W1, degout_p)


def _k4_body(agg_ref, dego_ref, degi_ref, b1_ref, w2_ref, out_ref):
    ns = _rsqrt_deg(dego_ref)
    nd = _rsqrt_deg(degi_ref)
    t = None
    for q in range(4):
        a = jnp.maximum(agg_ref[q] * nd + b1_ref[q:q + 1, :], 0.0)
        aq = jnp.dot(a, w2_ref[64 * q:64 * q + 64],
                     preferred_element_type=jnp.float32)
        t = aq if t is None else t + aq
    out_ref[...] = t * ns


def _k4(agg1_p, degout_p, degin_p, b1r, W2):
    return pl.pallas_call(
        _k4_body,
        grid=(10,),
        in_specs=[
            pl.BlockSpec((4, _RB, 64), lambda i: (0, i, 0)),
            pl.BlockSpec((2, _RB, 16), lambda i: (0, i, 0)),
            pl.BlockSpec((2, _RB, 16), lambda i: (0, i, 0)),
            pl.BlockSpec((4, 64), lambda i: (0, 0)),
            pl.BlockSpec((256, 64), lambda i: (0, 0)),
        ],
        out_specs=pl.BlockSpec((_RB, 64), lambda i: (i, 0)),
        out_shape=jax.ShapeDtypeStruct((NP, 64), jnp.float32),
    )(agg1_p, degout_p, degin_p, b1r, W2)


def _k6_body(agg_ref, degi_ref, b2_ref, wp1_ref, bp1_ref, a_ref, b_ref):
    nd = _rsqrt_deg(degi_ref)
    h2 = (agg_ref[0] + agg_ref[1]) * nd + b2_ref[0:1, :]
    a_ref[...] = (jnp.dot(h2, wp1_ref[0:64], preferred_element_type=jnp.float32)
                  + bp1_ref[0:1, :])
    b_ref[...] = jnp.dot(h2, wp1_ref[64:128], preferred_element_type=jnp.float32)


def _k6(agg2_p, degin_p, b2r, Wp1, bp1r):
    return pl.pallas_call(
        _k6_body,
        grid=(10,),
        in_specs=[
            pl.BlockSpec((2, _RB, 64), lambda i: (0, i, 0)),
            pl.BlockSpec((2, _RB, 16), lambda i: (0, i, 0)),
            pl.BlockSpec((1, 64), lambda i: (0, 0)),
            pl.BlockSpec((128, 64), lambda i: (0, 0)),
            pl.BlockSpec((1, 64), lambda i: (0, 0)),
        ],
        out_specs=[
            pl.BlockSpec((_RB, 64), lambda i: (i, 0)),
            pl.BlockSpec((_RB, 64), lambda i: (i, 0)),
        ],
        out_shape=[jax.ShapeDtypeStruct((NP, 64), jnp.float32),
                   jax.ShapeDtypeStruct((NP, 64), jnp.float32)],
    )(agg2_p, degin_p, b2r, Wp1, bp1r)


def _k8_body(c_ref, wp2_ref, bp2_ref, out_ref):
    z = jnp.maximum(c_ref[...], 0.0)
    out_ref[...] = (jnp.sum(z * wp2_ref[0:1, :], axis=1, keepdims=True)
                    + bp2_ref[0:1, :])


def _k8(C, wp2r, bp2r):
    blk = 4096
    return pl.pallas_call(
        _k8_body,
        grid=((2 * EP) // blk,),
        in_specs=[
            pl.BlockSpec((blk, 64), lambda i: (i, 0)),
            pl.BlockSpec((1, 64), lambda i: (0, 0)),
            pl.BlockSpec((1, 1), lambda i: (0, 0)),
        ],
        out_specs=pl.BlockSpec((blk, 1), lambda i: (i, 0)),
        out_shape=jax.ShapeDtypeStruct((2 * EP, 1), jnp.float32),
    )(C, wp2r, bp2r)


def kernel(x, edge_index, pos_edge_index, neg_edge_index,
           W1, b1, W2, b2, Wp1, bp1, Wp2, bp2):
    i32 = jnp.int32
    epad = jnp.full((EP - NEDGE,), NP - 1, i32)
    src_p = jnp.concatenate([edge_index[0], epad]).reshape(ROWS_E, CHUNK)
    dst_p = jnp.concatenate([edge_index[1], epad]).reshape(ROWS_E, CHUNK)
    src64 = src_p.reshape(EP // 64, 64)
    dst64 = dst_p.reshape(EP // 64, 64)

    zpad = jnp.zeros((EP - NEDGE,), i32)
    u_all = jnp.concatenate([pos_edge_index[0], zpad,
                             neg_edge_index[0], zpad]).reshape(-1, _PRED_CHUNK)
    v_all = jnp.concatenate([pos_edge_index[1], zpad,
                             neg_edge_index[1], zpad]).reshape(-1, _PRED_CHUNK)

    ones16 = jnp.ones((CHUNK, 16), jnp.float32)
    zeros16 = jnp.zeros((RP, 16), jnp.float32)
    zeros64 = jnp.zeros((RP, 64), jnp.float32)

    degout_p, degin_p = _deg_call(src_p, dst_p, ones16, zeros16)

    w1q = W1.reshape(512, 4, 64).transpose(1, 0, 2)
    h1s = _k2(x, w1q, degout_p)
    agg1_p = _agg1_call(src64, dst64, h1s, zeros64)

    h2in = _k4(agg1_p, degout_p, degin_p, b1.reshape(4, 64), W2)
    agg2_p = _agg2_call(src64, dst64, h2in.reshape(1, NP, 64), zeros64)

    A, B = _k6(agg2_p, degin_p, b2.reshape(1, 64), Wp1, bp1.reshape(1, 64))
    C = _pred_call(u_all, v_all, A, B)

    scores = _k8(C, Wp2.reshape(1, 64), bp2.reshape(1, 1))
    pos = scores[:NEDGE, 0]
    neg = scores[EP:EP + NEDGE, 0]
    return (pos, neg)


# speedup vs baseline: 6.3963x; 1.1045x over previous
import functools

import jax
import jax.numpy as jnp
from jax import lax
from jax.experimental import pallas as pl
from jax.experimental.pallas import tpu as pltpu
from jax.experimental.pallas import tpu_sc as plsc

NNODE = 10000
NP = 10240
NEDGE = 160000
CHUNK = 40
NC, NS = 2, 16
RP = NP // NS

_MESH = plsc.VectorSubcoreMesh(core_axis_name="c", subcore_axis_name="s")
_SC_PARAMS = pltpu.CompilerParams(use_tc_tiling_on_sc=False)


_DEG_ROWS = NEDGE // (NC * NS) // CHUNK


def _deg_body(e_h, ones_h, zeros_h, outp_h, inp_h,
              sidx2, didx2, ones_v, shout, shin, sem_a, sem_b):
    c = lax.axis_index("c")
    s = lax.axis_index("s")
    rbase = (c * NS + s) * _DEG_ROWS
    pltpu.sync_copy(e_h.at[0, pl.ds(rbase, _DEG_ROWS)], sidx2)
    pltpu.sync_copy(e_h.at[1, pl.ds(rbase, _DEG_ROWS)], didx2)
    pltpu.sync_copy(ones_h, ones_v)
    rows = pl.ds(s * RP, RP)
    pltpu.sync_copy(zeros_h, shout.at[rows])
    pltpu.sync_copy(zeros_h, shin.at[rows])
    plsc.subcore_barrier()

    @pl.loop(0, _DEG_ROWS, step=5)
    def _grp(i):
        hs = []
        for k in range(5):
            hs.append(pltpu.async_copy(ones_v, shout.at[sidx2.at[i + k]],
                                       sem_a, add=True))
            hs.append(pltpu.async_copy(ones_v, shin.at[didx2.at[i + k]],
                                       sem_b, add=True))
        for h in hs:
            h.wait()

    plsc.subcore_barrier()
    pltpu.sync_copy(shout.at[rows], outp_h.at[c, rows])
    pltpu.sync_copy(shin.at[rows], inp_h.at[c, rows])


_deg_call = functools.partial(
    pl.kernel,
    out_type=[jax.ShapeDtypeStruct((NC, NP, 16), jnp.float32),
              jax.ShapeDtypeStruct((NC, NP, 16), jnp.float32)],
    mesh=_MESH,
    compiler_params=_SC_PARAMS,
    scratch_types=[
        pltpu.VMEM((_DEG_ROWS, CHUNK), jnp.int32),
        pltpu.VMEM((_DEG_ROWS, CHUNK), jnp.int32),
        pltpu.VMEM((CHUNK, 16), jnp.float32),
        pltpu.VMEM_SHARED((NP, 16), jnp.float32),
        pltpu.VMEM_SHARED((NP, 16), jnp.float32),
        pltpu.SemaphoreType.DMA,
        pltpu.SemaphoreType.DMA,
    ],
)(_deg_body)


def _make_agg(passes, per_sc_edges_split, nbuf):
    if per_sc_edges_split:
        nrows = NEDGE // (NC * NS) // CHUNK
    else:
        nrows = NEDGE // NS // CHUNK
    nout = NC if per_sc_edges_split else passes * NC

    def body(e_h, h_h, zeros_h, agg_h, sidx2, didx2, *rest):
        bufs = rest[:nbuf]
        table = rest[nbuf]
        acc = rest[nbuf + 1]
        gsem = rest[nbuf + 2:2 * nbuf + 2]
        ssem = rest[2 * nbuf + 2:3 * nbuf + 2]
        c = lax.axis_index("c")
        s = lax.axis_index("s")
        if per_sc_edges_split:
            rbase = (c * NS + s) * nrows
        else:
            rbase = s * nrows
        rows = pl.ds(s * RP, RP)
        pltpu.sync_copy(e_h.at[0, pl.ds(rbase, nrows)], sidx2)
        pltpu.sync_copy(e_h.at[1, pl.ds(rbase, nrows)], didx2)

        for q in range(passes):
            qidx = 0 if per_sc_edges_split else c * passes + q
            pltpu.sync_copy(h_h.at[qidx, rows], table.at[rows])
            pltpu.sync_copy(zeros_h, acc.at[rows])
            plsc.subcore_barrier()

            @pl.loop(0, nrows, step=nbuf)
            def _grp(i):
                hg = [pltpu.async_copy(table.at[sidx2.at[i + k]], bufs[k],
                                       gsem[k]) for k in range(nbuf)]
                hs = []
                for k in range(nbuf):
                    hg[k].wait()
                    hs.append(pltpu.async_copy(bufs[k],
                                               acc.at[didx2.at[i + k]],
                                               ssem[k], add=True))
                for h in hs:
                    h.wait()

            plsc.subcore_barrier()
            out_idx = c if per_sc_edges_split else c * passes + q
            pltpu.sync_copy(acc.at[rows], agg_h.at[out_idx, rows])

    return functools.partial(
        pl.kernel,
        out_type=jax.ShapeDtypeStruct((nout, NP, 64), jnp.float32),
        mesh=_MESH,
        compiler_params=_SC_PARAMS,
        scratch_types=(
            [pltpu.VMEM((nrows, CHUNK), jnp.int32),
             pltpu.VMEM((nrows, CHUNK), jnp.int32)]
            + [pltpu.VMEM((CHUNK, 64), jnp.float32)] * nbuf
            + [pltpu.VMEM_SHARED((NP, 64), jnp.float32),
               pltpu.VMEM_SHARED((NP, 64), jnp.float32)]
            + [pltpu.SemaphoreType.DMA] * (2 * nbuf)
        ),
    )(body)


_agg1_call = _make_agg(passes=2, per_sc_edges_split=False, nbuf=5)
_agg2_call = _make_agg(passes=1, per_sc_edges_split=True, nbuf=5)


_PRED_NBUF = 5
_PRED_ROWS = NEDGE // (NC * NS) // CHUNK


def _pred_body(pe_h, ne_h, a_h, b_h, cp_h, cn_h, uidx2, vidx2, *rest):
    c = lax.axis_index("c")
    s = lax.axis_index("s")
    nbuf = _PRED_NBUF
    bufs = rest[:nbuf]
    sha = rest[nbuf]
    shb = rest[nbuf + 1]
    asem = rest[nbuf + 2:2 * nbuf + 2]
    bsem = rest[2 * nbuf + 2:3 * nbuf + 2]
    osem = rest[3 * nbuf + 2:4 * nbuf + 2]
    rbase = (c * NS + s) * _PRED_ROWS
    rows = pl.ds(s * RP, RP)
    pltpu.sync_copy(a_h.at[rows], sha.at[rows])
    pltpu.sync_copy(b_h.at[rows], shb.at[rows])
    plsc.subcore_barrier()

    for e_h, c_h in ((pe_h, cp_h), (ne_h, cn_h)):
        pltpu.sync_copy(e_h.at[0, pl.ds(rbase, _PRED_ROWS)], uidx2)
        pltpu.sync_copy(e_h.at[1, pl.ds(rbase, _PRED_ROWS)], vidx2)

        @pl.loop(0, _PRED_ROWS, step=nbuf)
        def _grp(i):
            ha = [pltpu.async_copy(sha.at[uidx2.at[i + k]], bufs[k], asem[k])
                  for k in range(nbuf)]
            hb = []
            for k in range(nbuf):
                ha[k].wait()
                hb.append(pltpu.async_copy(shb.at[vidx2.at[i + k]], bufs[k],
                                           bsem[k], add=True))
            ho = []
            for k in range(nbuf):
                hb[k].wait()
                row0 = (rbase + i + k) * CHUNK
                ho.append(pltpu.async_copy(bufs[k],
                                           c_h.at[pl.ds(row0, CHUNK)],
                                           osem[k]))
            for h in ho:
                h.wait()


_pred_call = functools.partial(
    pl.kernel,
    out_type=[jax.ShapeDtypeStruct((NEDGE, 64), jnp.float32),
              jax.ShapeDtypeStruct((NEDGE, 64), jnp.float32)],
    mesh=_MESH,
    compiler_params=_SC_PARAMS,
    scratch_types=(
        [pltpu.VMEM((_PRED_ROWS, CHUNK), jnp.int32),
         pltpu.VMEM((_PRED_ROWS, CHUNK), jnp.int32)]
        + [pltpu.VMEM((CHUNK, 64), jnp.float32)] * _PRED_NBUF
        + [pltpu.VMEM_SHARED((NP, 64), jnp.float32),
           pltpu.VMEM_SHARED((NP, 64), jnp.float32)]
        + [pltpu.SemaphoreType.DMA] * (3 * _PRED_NBUF)
    ),
)(_pred_body)


_RB = 1000


def _rsqrt_deg(ref):
    d = ref[0, :, 0:1] + ref[1, :, 0:1]
    return lax.rsqrt(jnp.maximum(d, 1.0))


def _k2_body(x_ref, w_ref, dego_ref, out_ref):
    ns = _rsqrt_deg(dego_ref)
    xw = jnp.dot(x_ref[...], w_ref[0], preferred_element_type=jnp.float32)
    out_ref[0] = xw * ns


def _k2(x, W1q, degout_p):
    return pl.pallas_call(
        _k2_body,
        grid=(10, 4),
        in_specs=[
            pl.BlockSpec((_RB, 512), lambda i, h: (i, 0)),
            pl.BlockSpec((1, 512, 64), lambda i, h: (h, 0, 0)),
            pl.BlockSpec((2, _RB, 16), lambda i, h: (0, i, 0)),
        ],
        out_specs=pl.BlockSpec((1, _RB, 64), lambda i, h: (h, i, 0)),
        out_shape=jax.ShapeDtypeStruct((4, NP, 64), jnp.float32),
    )(x, W1q, degout_p)


def _k4_body(agg_ref, dego_ref, degi_ref, b1_ref, w2_ref, out_ref):
    ns = _rsqrt_deg(dego_ref)
    nd = _rsqrt_deg(degi_ref)
    t = None
    for q in range(4):
        a = jnp.maximum(agg_ref[q] * nd + b1_ref[q:q + 1, :], 0.0)
        aq = jnp.dot(a, w2_ref[64 * q:64 * q + 64],
                     preferred_element_type=jnp.float32)
        t = aq if t is None else t + aq
    out_ref[...] = t * ns


def _k4(agg1_p, degout_p, degin_p, b1r, W2):
    return pl.pallas_call(
        _k4_body,
        grid=(10,),
        in_specs=[
            pl.BlockSpec((4, _RB, 64), lambda i: (0, i, 0)),
            pl.BlockSpec((2, _RB, 16), lambda i: (0, i, 0)),
            pl.BlockSpec((2, _RB, 16), lambda i: (0, i, 0)),
            pl.BlockSpec((4, 64), lambda i: (0, 0)),
            pl.BlockSpec((256, 64), lambda i: (0, 0)),
        ],
        out_specs=pl.BlockSpec((_RB, 64), lambda i: (i, 0)),
        out_shape=jax.ShapeDtypeStruct((NP, 64), jnp.float32),
    )(agg1_p, degout_p, degin_p, b1r, W2)


def _k6_body(agg_ref, degi_ref, b2_ref, wp1_ref, bp1_ref, a_ref, b_ref):
    nd = _rsqrt_deg(degi_ref)
    h2 = (agg_ref[0] + agg_ref[1]) * nd + b2_ref[0:1, :]
    a_ref[...] = (jnp.dot(h2, wp1_ref[0:64], preferred_element_type=jnp.float32)
                  + bp1_ref[0:1, :])
    b_ref[...] = jnp.dot(h2, wp1_ref[64:128], preferred_element_type=jnp.float32)


def _k6(agg2_p, degin_p, b2r, Wp1, bp1r):
    return pl.pallas_call(
        _k6_body,
        grid=(10,),
        in_specs=[
            pl.BlockSpec((2, _RB, 64), lambda i: (0, i, 0)),
            pl.BlockSpec((2, _RB, 16), lambda i: (0, i, 0)),
            pl.BlockSpec((1, 64), lambda i: (0, 0)),
            pl.BlockSpec((128, 64), lambda i: (0, 0)),
            pl.BlockSpec((1, 64), lambda i: (0, 0)),
        ],
        out_specs=[
            pl.BlockSpec((_RB, 64), lambda i: (i, 0)),
            pl.BlockSpec((_RB, 64), lambda i: (i, 0)),
        ],
        out_shape=[jax.ShapeDtypeStruct((NP, 64), jnp.float32),
                   jax.ShapeDtypeStruct((NP, 64), jnp.float32)],
    )(agg2_p, degin_p, b2r, Wp1, bp1r)


def _k8_body(cp_ref, cn_ref, wp2_ref, bp2_ref, op_ref, on_ref):
    zp = jnp.maximum(cp_ref[...], 0.0)
    zn = jnp.maximum(cn_ref[...], 0.0)
    op_ref[...] = (jnp.sum(zp * wp2_ref[0:1, :], axis=1, keepdims=True)
                   + bp2_ref[0:1, :])
    on_ref[...] = (jnp.sum(zn * wp2_ref[0:1, :], axis=1, keepdims=True)
                   + bp2_ref[0:1, :])


def _k8(Cp, Cn, wp2r, bp2r):
    blk = 4000
    return pl.pallas_call(
        _k8_body,
        grid=(NEDGE // blk,),
        in_specs=[
            pl.BlockSpec((blk, 64), lambda i: (i, 0)),
            pl.BlockSpec((blk, 64), lambda i: (i, 0)),
            pl.BlockSpec((1, 64), lambda i: (0, 0)),
            pl.BlockSpec((1, 1), lambda i: (0, 0)),
        ],
        out_specs=[
            pl.BlockSpec((blk, 1), lambda i: (i, 0)),
            pl.BlockSpec((blk, 1), lambda i: (i, 0)),
        ],
        out_shape=[jax.ShapeDtypeStruct((NEDGE, 1), jnp.float32),
                   jax.ShapeDtypeStruct((NEDGE, 1), jnp.float32)],
    )(Cp, Cn, wp2r, bp2r)


def kernel(x, edge_index, pos_edge_index, neg_edge_index,
           W1, b1, W2, b2, Wp1, bp1, Wp2, bp2):
    ones16 = jnp.ones((CHUNK, 16), jnp.float32)
    zeros16 = jnp.zeros((RP, 16), jnp.float32)
    zeros64 = jnp.zeros((RP, 64), jnp.float32)

    e3 = edge_index.reshape(2, NEDGE // CHUNK, CHUNK)
    p3 = pos_edge_index.reshape(2, NEDGE // CHUNK, CHUNK)
    n3 = neg_edge_index.reshape(2, NEDGE // CHUNK, CHUNK)

    degout_p, degin_p = _deg_call(e3, ones16, zeros16)

    w1q = W1.reshape(512, 4, 64).transpose(1, 0, 2)
    h1s = _k2(x, w1q, degout_p)
    agg1_p = _agg1_call(e3, h1s, zeros64)

    h2in = _k4(agg1_p, degout_p, degin_p, b1.reshape(4, 64), W2)
    agg2_p = _agg2_call(e3, h2in.reshape(1, NP, 64), zeros64)

    A, B = _k6(agg2_p, degin_p, b2.reshape(1, 64), Wp1, bp1.reshape(1, 64))
    Cp, Cn = _pred_call(p3, n3, A, B)

    pos, neg = _k8(Cp, Cn, Wp2.reshape(1, 64), bp2.reshape(1, 1))
    return (pos.reshape(NEDGE), neg.reshape(NEDGE))


# speedup vs baseline: 9.9448x; 1.5548x over previous
import functools

import jax
import jax.numpy as jnp
from jax import lax
from jax.experimental import pallas as pl
from jax.experimental.pallas import tpu as pltpu
from jax.experimental.pallas import tpu_sc as plsc

NNODE = 10000
NP = 10240
NEDGE = 160000
CHUNK = 40
NC, NS = 2, 16
RP = NP // NS

_MESH = plsc.VectorSubcoreMesh(core_axis_name="c", subcore_axis_name="s")
_SC_PARAMS = pltpu.CompilerParams(use_tc_tiling_on_sc=False)


_DEG_ROWS = NEDGE // (NC * NS) // CHUNK


def _deg_body(e_h, ones_h, zeros_h, outp_h, inp_h,
              sidx2, didx2, ones_v, shout, shin, sem_a, sem_b):
    c = lax.axis_index("c")
    s = lax.axis_index("s")
    rbase = (c * NS + s) * _DEG_ROWS
    pltpu.sync_copy(e_h.at[0, pl.ds(rbase, _DEG_ROWS)], sidx2)
    pltpu.sync_copy(e_h.at[1, pl.ds(rbase, _DEG_ROWS)], didx2)
    pltpu.sync_copy(ones_h, ones_v)
    rows = pl.ds(s * RP, RP)
    pltpu.sync_copy(zeros_h, shout.at[rows])
    pltpu.sync_copy(zeros_h, shin.at[rows])
    plsc.subcore_barrier()

    @pl.loop(0, _DEG_ROWS, step=5)
    def _grp(i):
        hs = []
        for k in range(5):
            hs.append(pltpu.async_copy(ones_v, shout.at[sidx2.at[i + k]],
                                       sem_a, add=True))
            hs.append(pltpu.async_copy(ones_v, shin.at[didx2.at[i + k]],
                                       sem_b, add=True))
        for h in hs:
            h.wait()

    plsc.subcore_barrier()
    pltpu.sync_copy(shout.at[rows], outp_h.at[c, rows])
    pltpu.sync_copy(shin.at[rows], inp_h.at[c, rows])


_deg_call = functools.partial(
    pl.kernel,
    out_type=[jax.ShapeDtypeStruct((NC, NP, 16), jnp.float32),
              jax.ShapeDtypeStruct((NC, NP, 16), jnp.float32)],
    mesh=_MESH,
    compiler_params=_SC_PARAMS,
    scratch_types=[
        pltpu.VMEM((_DEG_ROWS, CHUNK), jnp.int32),
        pltpu.VMEM((_DEG_ROWS, CHUNK), jnp.int32),
        pltpu.VMEM((CHUNK, 16), jnp.float32),
        pltpu.VMEM_SHARED((NP, 16), jnp.float32),
        pltpu.VMEM_SHARED((NP, 16), jnp.float32),
        pltpu.SemaphoreType.DMA,
        pltpu.SemaphoreType.DMA,
    ],
)(_deg_body)


def _make_agg(passes, per_sc_edges_split, nbuf):
    if per_sc_edges_split:
        nrows = NEDGE // (NC * NS) // CHUNK
    else:
        nrows = NEDGE // NS // CHUNK
    nout = NC if per_sc_edges_split else passes * NC

    def body(e_h, h_h, zeros_h, agg_h, sidx2, didx2, *rest):
        bufs = rest[:nbuf]
        table = rest[nbuf]
        acc = rest[nbuf + 1]
        gsem = rest[nbuf + 2:2 * nbuf + 2]
        ssem = rest[2 * nbuf + 2:3 * nbuf + 2]
        c = lax.axis_index("c")
        s = lax.axis_index("s")
        if per_sc_edges_split:
            rbase = (c * NS + s) * nrows
        else:
            rbase = s * nrows
        rows = pl.ds(s * RP, RP)
        pltpu.sync_copy(e_h.at[0, pl.ds(rbase, nrows)], sidx2)
        pltpu.sync_copy(e_h.at[1, pl.ds(rbase, nrows)], didx2)

        for q in range(passes):
            qidx = 0 if per_sc_edges_split else c * passes + q
            pltpu.sync_copy(h_h.at[qidx, rows], table.at[rows])
            pltpu.sync_copy(zeros_h, acc.at[rows])
            plsc.subcore_barrier()

            @pl.loop(0, nrows, step=nbuf)
            def _grp(i):
                hg = [pltpu.async_copy(table.at[sidx2.at[i + k]], bufs[k],
                                       gsem[k]) for k in range(nbuf)]
                hs = []
                for k in range(nbuf):
                    hg[k].wait()
                    hs.append(pltpu.async_copy(bufs[k],
                                               acc.at[didx2.at[i + k]],
                                               ssem[k], add=True))
                for h in hs:
                    h.wait()

            plsc.subcore_barrier()
            out_idx = c if per_sc_edges_split else c * passes + q
            pltpu.sync_copy(acc.at[rows], agg_h.at[out_idx, rows])

    return functools.partial(
        pl.kernel,
        out_type=jax.ShapeDtypeStruct((nout, NP, 64), jnp.float32),
        mesh=_MESH,
        compiler_params=_SC_PARAMS,
        scratch_types=(
            [pltpu.VMEM((nrows, CHUNK), jnp.int32),
             pltpu.VMEM((nrows, CHUNK), jnp.int32)]
            + [pltpu.VMEM((CHUNK, 64), jnp.float32)] * nbuf
            + [pltpu.VMEM_SHARED((NP, 64), jnp.float32),
               pltpu.VMEM_SHARED((NP, 64), jnp.float32)]
            + [pltpu.SemaphoreType.DMA] * (2 * nbuf)
        ),
    )(body)


_agg1_call = _make_agg(passes=2, per_sc_edges_split=False, nbuf=5)
_agg2_call = _make_agg(passes=1, per_sc_edges_split=True, nbuf=5)


_PRED_NBUF = 5
_PRED_ROWS = NEDGE // (NC * NS) // CHUNK


def _pred_body(pe_h, ne_h, a_h, b_h, w_h, sp_h, sn_h, uidx2, vidx2, *rest):
    c = lax.axis_index("c")
    s = lax.axis_index("s")
    nbuf = _PRED_NBUF
    bufs = rest[:nbuf]
    obufs = rest[nbuf:2 * nbuf]
    wv = rest[2 * nbuf]
    sha = rest[2 * nbuf + 1]
    shb = rest[2 * nbuf + 2]
    asem = rest[2 * nbuf + 3:3 * nbuf + 3]
    bsem = rest[3 * nbuf + 3:4 * nbuf + 3]
    osem = rest[4 * nbuf + 3:5 * nbuf + 3]
    rbase = (c * NS + s) * _PRED_ROWS
    rows = pl.ds(s * RP, RP)
    pltpu.sync_copy(a_h.at[rows], sha.at[rows])
    pltpu.sync_copy(b_h.at[rows], shb.at[rows])
    pltpu.sync_copy(w_h, wv)
    w0 = wv[pl.ds(0, 16)]
    w1 = wv[pl.ds(16, 16)]
    w2 = wv[pl.ds(32, 16)]
    w3 = wv[pl.ds(48, 16)]
    bp2s = wv[pl.ds(64, 16)][0]
    lane = lax.iota(jnp.int32, 16)
    plsc.subcore_barrier()

    for e_h, o_h in ((pe_h, sp_h), (ne_h, sn_h)):
        pltpu.sync_copy(e_h.at[0, pl.ds(rbase, _PRED_ROWS)], uidx2)
        pltpu.sync_copy(e_h.at[1, pl.ds(rbase, _PRED_ROWS)], vidx2)

        @pl.loop(0, _PRED_ROWS, step=nbuf)
        def _grp(i):
            ha = [pltpu.async_copy(sha.at[uidx2.at[i + k]],
                                   bufs[k].at[pl.ds(0, CHUNK)], asem[k])
                  for k in range(nbuf)]
            hb = []
            for k in range(nbuf):
                ha[k].wait()
                hb.append(pltpu.async_copy(shb.at[vidx2.at[i + k]],
                                           bufs[k].at[pl.ds(0, CHUNK)],
                                           bsem[k], add=True))
            ho = []
            for k in range(nbuf):
                hb[k].wait()
                buf = bufs[k]
                obuf = obufs[k]

                @pl.loop(0, 3)
                def _egrp(g):
                    z = jnp.float32(0.0)
                    svec = jnp.zeros((16,), jnp.float32)
                    for j in range(16):
                        e = g * 16 + j
                        acc = (jnp.maximum(buf[e, 0:16], z) * w0
                               + jnp.maximum(buf[e, 16:32], z) * w1
                               + jnp.maximum(buf[e, 32:48], z) * w2
                               + jnp.maximum(buf[e, 48:64], z) * w3)
                        sval = jnp.sum(acc) + bp2s
                        svec = jnp.where(lane == j, sval, svec)
                    obuf[pl.ds(g * 16, 16)] = svec

                ho.append(pltpu.async_copy(
                    obuf.at[pl.ds(0, CHUNK)],
                    o_h.at[pl.ds((rbase + i + k) * CHUNK, CHUNK)],
                    osem[k]))
            for h in ho:
                h.wait()


_pred_call = functools.partial(
    pl.kernel,
    out_type=[jax.ShapeDtypeStruct((NEDGE,), jnp.float32),
              jax.ShapeDtypeStruct((NEDGE,), jnp.float32)],
    mesh=_MESH,
    compiler_params=pltpu.CompilerParams(use_tc_tiling_on_sc=False,
                                         needs_layout_passes=False),
    scratch_types=(
        [pltpu.VMEM((_PRED_ROWS, CHUNK), jnp.int32),
         pltpu.VMEM((_PRED_ROWS, CHUNK), jnp.int32)]
        + [pltpu.VMEM((48, 64), jnp.float32)] * _PRED_NBUF
        + [pltpu.VMEM((48,), jnp.float32)] * _PRED_NBUF
        + [pltpu.VMEM((80,), jnp.float32),
           pltpu.VMEM_SHARED((NP, 64), jnp.float32),
           pltpu.VMEM_SHARED((NP, 64), jnp.float32)]
        + [pltpu.SemaphoreType.DMA] * (3 * _PRED_NBUF)
    ),
)(_pred_body)


_RB = 1000


def _rsqrt_deg(ref):
    d = ref[0, :, 0:1] + ref[1, :, 0:1]
    return lax.rsqrt(jnp.maximum(d, 1.0))


def _k2_body(x_ref, w_ref, dego_ref, out_ref):
    ns = _rsqrt_deg(dego_ref)
    xw = jnp.dot(x_ref[...], w_ref[0], preferred_element_type=jnp.float32)
    out_ref[0] = xw * ns


def _k2(x, W1q, degout_p):
    return pl.pallas_call(
        _k2_body,
        grid=(10, 4),
        in_specs=[
            pl.BlockSpec((_RB, 512), lambda i, h: (i, 0)),
            pl.BlockSpec((1, 512, 64), lambda i, h: (h, 0, 0)),
            pl.BlockSpec((2, _RB, 16), lambda i, h: (0, i, 0)),
        ],
        out_specs=pl.BlockSpec((1, _RB, 64), lambda i, h: (h, i, 0)),
        out_shape=jax.ShapeDtypeStruct((4, NP, 64), jnp.float32),
    )(x, W1q, degout_p)


def _k4_body(agg_ref, dego_ref, degi_ref, b1_ref, w2_ref, out_ref):
    ns = _rsqrt_deg(dego_ref)
    nd = _rsqrt_deg(degi_ref)
    t = None
    for q in range(4):
        a = jnp.maximum(agg_ref[q] * nd + b1_ref[q:q + 1, :], 0.0)
        aq = jnp.dot(a, w2_ref[64 * q:64 * q + 64],
                     preferred_element_type=jnp.float32)
        t = aq if t is None else t + aq
    out_ref[...] = t * ns


def _k4(agg1_p, degout_p, degin_p, b1r, W2):
    return pl.pallas_call(
        _k4_body,
        grid=(10,),
        in_specs=[
            pl.BlockSpec((4, _RB, 64), lambda i: (0, i, 0)),
            pl.BlockSpec((2, _RB, 16), lambda i: (0, i, 0)),
            pl.BlockSpec((2, _RB, 16), lambda i: (0, i, 0)),
            pl.BlockSpec((4, 64), lambda i: (0, 0)),
            pl.BlockSpec((256, 64), lambda i: (0, 0)),
        ],
        out_specs=pl.BlockSpec((_RB, 64), lambda i: (i, 0)),
        out_shape=jax.ShapeDtypeStruct((NP, 64), jnp.float32),
    )(agg1_p, degout_p, degin_p, b1r, W2)


def _k6_body(agg_ref, degi_ref, b2_ref, wp1_ref, bp1_ref, a_ref, b_ref):
    nd = _rsqrt_deg(degi_ref)
    h2 = (agg_ref[0] + agg_ref[1]) * nd + b2_ref[0:1, :]
    a_ref[...] = (jnp.dot(h2, wp1_ref[0:64], preferred_element_type=jnp.float32)
                  + bp1_ref[0:1, :])
    b_ref[...] = jnp.dot(h2, wp1_ref[64:128], preferred_element_type=jnp.float32)


def _k6(agg2_p, degin_p, b2r, Wp1, bp1r):
    return pl.pallas_call(
        _k6_body,
        grid=(10,),
        in_specs=[
            pl.BlockSpec((2, _RB, 64), lambda i: (0, i, 0)),
            pl.BlockSpec((2, _RB, 16), lambda i: (0, i, 0)),
            pl.BlockSpec((1, 64), lambda i: (0, 0)),
            pl.BlockSpec((128, 64), lambda i: (0, 0)),
            pl.BlockSpec((1, 64), lambda i: (0, 0)),
        ],
        out_specs=[
            pl.BlockSpec((_RB, 64), lambda i: (i, 0)),
            pl.BlockSpec((_RB, 64), lambda i: (i, 0)),
        ],
        out_shape=[jax.ShapeDtypeStruct((NP, 64), jnp.float32),
                   jax.ShapeDtypeStruct((NP, 64), jnp.float32)],
    )(agg2_p, degin_p, b2r, Wp1, bp1r)


def kernel(x, edge_index, pos_edge_index, neg_edge_index,
           W1, b1, W2, b2, Wp1, bp1, Wp2, bp2):
    ones16 = jnp.ones((CHUNK, 16), jnp.float32)
    zeros16 = jnp.zeros((RP, 16), jnp.float32)
    zeros64 = jnp.zeros((RP, 64), jnp.float32)

    e3 = edge_index.reshape(2, NEDGE // CHUNK, CHUNK)
    p3 = pos_edge_index.reshape(2, NEDGE // CHUNK, CHUNK)
    n3 = neg_edge_index.reshape(2, NEDGE // CHUNK, CHUNK)

    degout_p, degin_p = _deg_call(e3, ones16, zeros16)

    w1q = W1.reshape(512, 4, 64).transpose(1, 0, 2)
    h1s = _k2(x, w1q, degout_p)
    agg1_p = _agg1_call(e3, h1s, zeros64)

    h2in = _k4(agg1_p, degout_p, degin_p, b1.reshape(4, 64), W2)
    agg2_p = _agg2_call(e3, h2in.reshape(1, NP, 64), zeros64)

    A, B = _k6(agg2_p, degin_p, b2.reshape(1, 64), Wp1, bp1.reshape(1, 64))
    wvec = jnp.concatenate([Wp2.reshape(64), bp2, jnp.zeros((15,), jnp.float32)])
    pos, neg = _pred_call(p3, n3, A, B, wvec)
    return (pos, neg)
